# Initial kernel scaffold; baseline (speedup 1.0000x reference)
import functools

import jax
import jax.numpy as jnp
from jax import lax
from jax.experimental import pallas as pl
from jax.experimental.pallas import tpu as pltpu
from jax.experimental.pallas import tpu_sc as plsc

N = 100000
E = 1600000
F = 165
H = 32
CLS = 2
EPS = 1e-5

NC = 2
NS = 16
NP = 100096
RP = NP // NS
RB = N // NS

CH = 2000
EPW_HALF = E // 2 // NS
EPW_ALL = E // NS

BM = 2000

_mesh = plsc.VectorSubcoreMesh(core_axis_name="c", subcore_axis_name="s")


@functools.partial(
    pl.kernel,
    out_type=jax.ShapeDtypeStruct((NC, NP), jnp.float32),
    mesh=_mesh,
    scratch_types=[
        pltpu.VMEM((CH,), jnp.int32),
        pltpu.VMEM((CH,), jnp.float32),
        pltpu.VMEM_SHARED((NP,), jnp.float32),
    ],
)
def _deg_kernel(dst_hbm, ones_hbm, zeros_hbm, out_hbm, idx_v, ones_v, acc_sh):
    cid = lax.axis_index("c")
    sid = lax.axis_index("s")
    pltpu.sync_copy(ones_hbm, ones_v)
    pltpu.sync_copy(zeros_hbm.at[pl.ds(sid * RP, RP)],
                    acc_sh.at[pl.ds(sid * RP, RP)])
    plsc.subcore_barrier()

    base = cid * (E // 2) + sid * EPW_HALF

    @pl.loop(0, EPW_HALF, step=CH)
    def _(e0):
        pltpu.sync_copy(dst_hbm.at[pl.ds(base + e0, CH)], idx_v)
        pltpu.sync_copy(ones_v, acc_sh.at[idx_v], add=True)

    plsc.subcore_barrier()
    pltpu.sync_copy(acc_sh.at[pl.ds(sid * RP, RP)],
                    out_hbm.at[cid, pl.ds(sid * RP, RP)])


@functools.partial(
    pl.kernel,
    out_type=jax.ShapeDtypeStruct((NC, N, 16), jnp.float32),
    mesh=_mesh,
    scratch_types=[
        pltpu.VMEM((CH,), jnp.int32),
        pltpu.VMEM((CH,), jnp.int32),
        pltpu.VMEM((CH, 16), jnp.float32),
        pltpu.VMEM_SHARED((N, 16), jnp.float32),
    ],
)
def _seg32_kernel(y3_hbm, src_hbm, dst_hbm, zeros_hbm, out_hbm,
                  si_v, di_v, rows_v, acc_sh):
    cid = lax.axis_index("c")
    sid = lax.axis_index("s")
    pltpu.sync_copy(zeros_hbm.at[pl.ds(sid * RB, RB)],
                    acc_sh.at[pl.ds(sid * RB, RB)])
    plsc.subcore_barrier()

    base = sid * EPW_ALL

    @pl.loop(0, EPW_ALL, step=CH)
    def _(e0):
        pltpu.sync_copy(src_hbm.at[pl.ds(base + e0, CH)], si_v)
        pltpu.sync_copy(dst_hbm.at[pl.ds(base + e0, CH)], di_v)
        pltpu.sync_copy(y3_hbm.at[cid].at[si_v], rows_v)
        pltpu.sync_copy(rows_v, acc_sh.at[di_v], add=True)

    plsc.subcore_barrier()
    pltpu.sync_copy(acc_sh.at[pl.ds(sid * RB, RB)],
                    out_hbm.at[cid, pl.ds(sid * RB, RB)])


@functools.partial(
    pl.kernel,
    out_type=jax.ShapeDtypeStruct((NC, NP, CLS), jnp.float32),
    mesh=_mesh,
    scratch_types=[
        pltpu.VMEM((CH,), jnp.int32),
        pltpu.VMEM((CH,), jnp.int32),
        pltpu.VMEM((CH, CLS), jnp.float32),
        pltpu.VMEM_SHARED((NP, CLS), jnp.float32),
    ],
)
def _seg2_kernel(y2_hbm, src_hbm, dst_hbm, zeros_hbm, out_hbm,
                 si_v, di_v, rows_v, acc_sh):
    cid = lax.axis_index("c")
    sid = lax.axis_index("s")
    pltpu.sync_copy(zeros_hbm.at[pl.ds(sid * RP, RP)],
                    acc_sh.at[pl.ds(sid * RP, RP)])
    plsc.subcore_barrier()

    base = cid * (E // 2) + sid * EPW_HALF

    @pl.loop(0, EPW_HALF, step=CH)
    def _(e0):
        pltpu.sync_copy(src_hbm.at[pl.ds(base + e0, CH)], si_v)
        pltpu.sync_copy(dst_hbm.at[pl.ds(base + e0, CH)], di_v)
        pltpu.sync_copy(y2_hbm.at[si_v], rows_v)
        pltpu.sync_copy(rows_v, acc_sh.at[di_v], add=True)

    plsc.subcore_barrier()
    pltpu.sync_copy(acc_sh.at[pl.ds(sid * RP, RP)],
                    out_hbm.at[cid, pl.ds(sid * RP, RP)])


def _m1_body(x_ref, w_ref, o_ref):
    o_ref[...] = jnp.dot(x_ref[...], w_ref[...],
                         preferred_element_type=jnp.float32)


def _s2_body(dp_ref, xw_ref, y3_ref, y32_ref, dis_ref):
    deg = dp_ref[:, 0:1] + dp_ref[:, 1:2] + 1.0
    dis = lax.rsqrt(deg)
    y = xw_ref[...] * dis
    y32_ref[...] = y
    dis_ref[...] = dis
    y3_ref[0] = y[:, :16]
    y3_ref[1] = y[:, 16:]


def _c2a_body(zp_ref, y_ref, dis_ref, b1_ref, t_ref, st_ref):
    i = pl.program_id(0)
    z = jnp.concatenate([zp_ref[0], zp_ref[1]], axis=1)
    t = dis_ref[...] * (z + y_ref[...]) + b1_ref[...]
    t_ref[...] = t

    @pl.when(i == 0)
    def _():
        st_ref[...] = jnp.zeros_like(st_ref)

    st_ref[0:1, :] += jnp.sum(t, axis=0, keepdims=True)
    st_ref[1:2, :] += jnp.sum(t * t, axis=0, keepdims=True)


def _c2b_body(t_ref, st_ref, g_ref, be_ref, dis_ref, w2_ref, y2_ref):
    mean = st_ref[0:1, :] * (1.0 / N)
    var = st_ref[1:2, :] * (1.0 / N) - mean * mean
    inv = lax.rsqrt(var + EPS)
    h = (t_ref[...] - mean) * inv * g_ref[...] + be_ref[...]
    h = jnp.maximum(h, 0.0)
    y2_ref[...] = jnp.dot(h, w2_ref[...],
                          preferred_element_type=jnp.float32) * dis_ref[...]


def _c3_body(z2p_ref, y2_ref, dis_ref, b2_ref, o_ref):
    z2 = z2p_ref[0] + z2p_ref[1]
    o = dis_ref[...] * (z2 + y2_ref[...]) + b2_ref[...]
    m = jnp.max(o, axis=1, keepdims=True)
    lse = m + jnp.log(jnp.sum(jnp.exp(o - m), axis=1, keepdims=True))
    o_ref[...] = o - lse


def kernel(x, edge_index, W1, b1, gamma, beta, W2, b2):
    ei = edge_index.astype(jnp.int32)
    src = ei[0]
    dst = ei[1]

    ones_chunk = jnp.ones((CH,), jnp.float32)
    zeros_np = jnp.zeros((NP,), jnp.float32)
    zeros_n16 = jnp.zeros((N, 16), jnp.float32)
    zeros_np2 = jnp.zeros((NP, CLS), jnp.float32)

    degp = _deg_kernel(dst, ones_chunk, zeros_np)

    xw = pl.pallas_call(
        _m1_body,
        grid=(N // BM,),
        in_specs=[
            pl.BlockSpec((BM, F), lambda i: (i, 0)),
            pl.BlockSpec((F, H), lambda i: (0, 0)),
        ],
        out_specs=pl.BlockSpec((BM, H), lambda i: (i, 0)),
        out_shape=jax.ShapeDtypeStruct((N, H), jnp.float32),
    )(x, W1)

    degp_t = jnp.transpose(degp)[:N]

    y3, y32, dis = pl.pallas_call(
        _s2_body,
        grid=(N // BM,),
        in_specs=[
            pl.BlockSpec((BM, NC), lambda i: (i, 0)),
            pl.BlockSpec((BM, H), lambda i: (i, 0)),
        ],
        out_specs=[
            pl.BlockSpec((NC, BM, 16), lambda i: (0, i, 0)),
            pl.BlockSpec((BM, H), lambda i: (i, 0)),
            pl.BlockSpec((BM, 1), lambda i: (i, 0)),
        ],
        out_shape=[
            jax.ShapeDtypeStruct((NC, N, 16), jnp.float32),
            jax.ShapeDtypeStruct((N, H), jnp.float32),
            jax.ShapeDtypeStruct((N, 1), jnp.float32),
        ],
    )(degp_t, xw)

    zp = _seg32_kernel(y3, src, dst, zeros_n16)

    t, st = pl.pallas_call(
        _c2a_body,
        grid=(N // BM,),
        in_specs=[
            pl.BlockSpec((NC, BM, 16), lambda i: (0, i, 0)),
            pl.BlockSpec((BM, H), lambda i: (i, 0)),
            pl.BlockSpec((BM, 1), lambda i: (i, 0)),
            pl.BlockSpec((1, H), lambda i: (0, 0)),
        ],
        out_specs=[
            pl.BlockSpec((BM, H), lambda i: (i, 0)),
            pl.BlockSpec((2, H), lambda i: (0, 0)),
        ],
        out_shape=[
            jax.ShapeDtypeStruct((N, H), jnp.float32),
            jax.ShapeDtypeStruct((2, H), jnp.float32),
        ],
    )(zp, y32, dis, b1.reshape(1, H))

    y2 = pl.pallas_call(
        _c2b_body,
        grid=(N // BM,),
        in_specs=[
            pl.BlockSpec((BM, H), lambda i: (i, 0)),
            pl.BlockSpec((2, H), lambda i: (0, 0)),
            pl.BlockSpec((1, H), lambda i: (0, 0)),
            pl.BlockSpec((1, H), lambda i: (0, 0)),
            pl.BlockSpec((BM, 1), lambda i: (i, 0)),
            pl.BlockSpec((H, CLS), lambda i: (0, 0)),
        ],
        out_specs=pl.BlockSpec((BM, CLS), lambda i: (i, 0)),
        out_shape=jax.ShapeDtypeStruct((N, CLS), jnp.float32),
    )(t, st, gamma.reshape(1, H), beta.reshape(1, H), dis, W2)

    z2p = _seg2_kernel(y2, src, dst, zeros_np2)

    out = pl.pallas_call(
        _c3_body,
        grid=(N // BM,),
        in_specs=[
            pl.BlockSpec((NC, BM, CLS), lambda i: (0, i, 0)),
            pl.BlockSpec((BM, CLS), lambda i: (i, 0)),
            pl.BlockSpec((BM, 1), lambda i: (i, 0)),
            pl.BlockSpec((1, CLS), lambda i: (0, 0)),
        ],
        out_specs=pl.BlockSpec((BM, CLS), lambda i: (i, 0)),
        out_shape=jax.ShapeDtypeStruct((N, CLS), jnp.float32),
    )(z2p, y2, dis, b2.reshape(1, CLS))

    return out


# speedup vs baseline: 19.3846x; 19.3846x over previous
import functools

import jax
import jax.numpy as jnp
from jax import lax
from jax.experimental import pallas as pl
from jax.experimental.pallas import tpu as pltpu
from jax.experimental.pallas import tpu_sc as plsc

N = 100000
E = 1600000
F = 165
H = 32
CLS = 2
EPS = 1e-5

NC = 2
NS = 16
NP = 100096
RP = NP // NS
RB = N // NS

CH = 1250
EPW_HALF = E // 2 // NS
EPW_ALL = E // NS

BM = 2000
NCHUNK = E // CH

_mesh = plsc.VectorSubcoreMesh(core_axis_name="c", subcore_axis_name="s")
_sc_params = pltpu.CompilerParams(use_tc_tiling_on_sc=False)


@functools.partial(
    pl.kernel,
    out_type=jax.ShapeDtypeStruct((NC, NP, 1), jnp.float32),
    mesh=_mesh,
    compiler_params=_sc_params,
    scratch_types=[
        pltpu.VMEM((1, CH), jnp.int32),
        pltpu.VMEM((CH, 1), jnp.float32),
        pltpu.VMEM_SHARED((NP, 1), jnp.float32),
    ],
)
def _deg_kernel(dst_hbm, ones_hbm, zeros_hbm, out_hbm, idx_v, ones_v, acc_sh):
    cid = lax.axis_index("c")
    sid = lax.axis_index("s")
    pltpu.sync_copy(ones_hbm, ones_v)
    pltpu.sync_copy(zeros_hbm.at[pl.ds(sid * RP, RP)],
                    acc_sh.at[pl.ds(sid * RP, RP)])
    plsc.subcore_barrier()

    nch = EPW_HALF // CH
    base = cid * (NCHUNK // 2) + sid * nch

    @pl.loop(0, nch)
    def _(k):
        pltpu.sync_copy(dst_hbm.at[base + k], idx_v)
        pltpu.sync_copy(ones_v, acc_sh.at[idx_v.at[0]], add=True)

    plsc.subcore_barrier()
    pltpu.sync_copy(acc_sh.at[pl.ds(sid * RP, RP)],
                    out_hbm.at[cid, pl.ds(sid * RP, RP)])


@functools.partial(
    pl.kernel,
    out_type=jax.ShapeDtypeStruct((NC, NP, 16), jnp.float32),
    mesh=_mesh,
    compiler_params=_sc_params,
    scratch_types=[
        pltpu.VMEM((1, CH), jnp.int32),
        pltpu.VMEM((1, CH), jnp.int32),
        pltpu.VMEM((CH, 16), jnp.float32),
        pltpu.VMEM_SHARED((NP, 16), jnp.float32),
    ],
)
def _seg32_kernel(y3_hbm, src_hbm, dst_hbm, zeros_hbm, out_hbm,
                  si_v, di_v, rows_v, acc_sh):
    cid = lax.axis_index("c")
    sid = lax.axis_index("s")
    pltpu.sync_copy(zeros_hbm.at[pl.ds(sid * RP, RP)],
                    acc_sh.at[pl.ds(sid * RP, RP)])
    plsc.subcore_barrier()

    nch = EPW_ALL // CH
    base = sid * nch

    @pl.loop(0, nch)
    def _(k):
        pltpu.sync_copy(src_hbm.at[base + k], si_v)
        pltpu.sync_copy(dst_hbm.at[base + k], di_v)
        pltpu.sync_copy(y3_hbm.at[cid].at[si_v.at[0]], rows_v)
        pltpu.sync_copy(rows_v, acc_sh.at[di_v.at[0]], add=True)

    plsc.subcore_barrier()
    pltpu.sync_copy(acc_sh.at[pl.ds(sid * RP, RP)],
                    out_hbm.at[cid, pl.ds(sid * RP, RP)])


@functools.partial(
    pl.kernel,
    out_type=jax.ShapeDtypeStruct((NC, NP, CLS), jnp.float32),
    mesh=_mesh,
    compiler_params=_sc_params,
    scratch_types=[
        pltpu.VMEM((1, CH), jnp.int32),
        pltpu.VMEM((1, CH), jnp.int32),
        pltpu.VMEM((CH, CLS), jnp.float32),
        pltpu.VMEM_SHARED((NP, CLS), jnp.float32),
    ],
)
def _seg2_kernel(y2_hbm, src_hbm, dst_hbm, zeros_hbm, out_hbm,
                 si_v, di_v, rows_v, acc_sh):
    cid = lax.axis_index("c")
    sid = lax.axis_index("s")
    pltpu.sync_copy(zeros_hbm.at[pl.ds(sid * RP, RP)],
                    acc_sh.at[pl.ds(sid * RP, RP)])
    plsc.subcore_barrier()

    nch = EPW_HALF // CH
    base = cid * (NCHUNK // 2) + sid * nch

    @pl.loop(0, nch)
    def _(k):
        pltpu.sync_copy(src_hbm.at[base + k], si_v)
        pltpu.sync_copy(dst_hbm.at[base + k], di_v)
        pltpu.sync_copy(y2_hbm.at[si_v.at[0]], rows_v)
        pltpu.sync_copy(rows_v, acc_sh.at[di_v.at[0]], add=True)

    plsc.subcore_barrier()
    pltpu.sync_copy(acc_sh.at[pl.ds(sid * RP, RP)],
                    out_hbm.at[cid, pl.ds(sid * RP, RP)])


def _m1_body(x_ref, w_ref, o_ref):
    o_ref[...] = jnp.dot(x_ref[...], w_ref[...],
                         preferred_element_type=jnp.float32)


def _s2_body(dp_ref, xw_ref, y3_ref, y32_ref, dis_ref):
    deg = dp_ref[:, 0:1] + dp_ref[:, 1:2] + 1.0
    dis = lax.rsqrt(deg)
    y = xw_ref[...] * dis
    y32_ref[...] = y
    dis_ref[...] = dis
    y3_ref[0] = y[:, :16]
    y3_ref[1] = y[:, 16:]


def _c2a_body(zp_ref, y_ref, dis_ref, b1_ref, t_ref, st_ref):
    i = pl.program_id(0)
    z = jnp.concatenate([zp_ref[0], zp_ref[1]], axis=1)
    t = dis_ref[...] * (z + y_ref[...]) + b1_ref[...]
    t_ref[...] = t

    @pl.when(i == 0)
    def _():
        st_ref[...] = jnp.zeros_like(st_ref)

    st_ref[0:1, :] += jnp.sum(t, axis=0, keepdims=True)
    st_ref[1:2, :] += jnp.sum(t * t, axis=0, keepdims=True)


def _c2b_body(t_ref, st_ref, g_ref, be_ref, dis_ref, w2_ref, y2_ref):
    mean = st_ref[0:1, :] * (1.0 / N)
    var = st_ref[1:2, :] * (1.0 / N) - mean * mean
    inv = lax.rsqrt(var + EPS)
    h = (t_ref[...] - mean) * inv * g_ref[...] + be_ref[...]
    h = jnp.maximum(h, 0.0)
    y2_ref[...] = jnp.dot(h, w2_ref[...],
                          preferred_element_type=jnp.float32) * dis_ref[...]


def _c3_body(z2p_ref, y2_ref, dis_ref, b2_ref, o_ref):
    z2 = z2p_ref[0] + z2p_ref[1]
    o = dis_ref[...] * (z2 + y2_ref[...]) + b2_ref[...]
    m = jnp.max(o, axis=1, keepdims=True)
    lse = m + jnp.log(jnp.sum(jnp.exp(o - m), axis=1, keepdims=True))
    o_ref[...] = o - lse


def kernel(x, edge_index, W1, b1, gamma, beta, W2, b2):
    ei = edge_index.astype(jnp.int32)
    src3 = ei[0].reshape(NCHUNK, 1, CH)
    dst3 = ei[1].reshape(NCHUNK, 1, CH)

    ones_chunk = jnp.ones((CH, 1), jnp.float32)
    zeros_np1 = jnp.zeros((NP, 1), jnp.float32)
    zeros_np16 = jnp.zeros((NP, 16), jnp.float32)
    zeros_np2 = jnp.zeros((NP, CLS), jnp.float32)

    degp = _deg_kernel(dst3, ones_chunk, zeros_np1)

    xw = pl.pallas_call(
        _m1_body,
        grid=(N // BM,),
        in_specs=[
            pl.BlockSpec((BM, F), lambda i: (i, 0)),
            pl.BlockSpec((F, H), lambda i: (0, 0)),
        ],
        out_specs=pl.BlockSpec((BM, H), lambda i: (i, 0)),
        out_shape=jax.ShapeDtypeStruct((N, H), jnp.float32),
    )(x, W1)

    degp_t = jnp.transpose(degp[:, :, 0])[:N]

    y3, y32, dis = pl.pallas_call(
        _s2_body,
        grid=(N // BM,),
        in_specs=[
            pl.BlockSpec((BM, NC), lambda i: (i, 0)),
            pl.BlockSpec((BM, H), lambda i: (i, 0)),
        ],
        out_specs=[
            pl.BlockSpec((NC, BM, 16), lambda i: (0, i, 0)),
            pl.BlockSpec((BM, H), lambda i: (i, 0)),
            pl.BlockSpec((BM, 1), lambda i: (i, 0)),
        ],
        out_shape=[
            jax.ShapeDtypeStruct((NC, N, 16), jnp.float32),
            jax.ShapeDtypeStruct((N, H), jnp.float32),
            jax.ShapeDtypeStruct((N, 1), jnp.float32),
        ],
    )(degp_t, xw)

    zp = _seg32_kernel(y3, src3, dst3, zeros_np16)

    t, st = pl.pallas_call(
        _c2a_body,
        grid=(N // BM,),
        in_specs=[
            pl.BlockSpec((NC, BM, 16), lambda i: (0, i, 0)),
            pl.BlockSpec((BM, H), lambda i: (i, 0)),
            pl.BlockSpec((BM, 1), lambda i: (i, 0)),
            pl.BlockSpec((1, H), lambda i: (0, 0)),
        ],
        out_specs=[
            pl.BlockSpec((BM, H), lambda i: (i, 0)),
            pl.BlockSpec((2, H), lambda i: (0, 0)),
        ],
        out_shape=[
            jax.ShapeDtypeStruct((N, H), jnp.float32),
            jax.ShapeDtypeStruct((2, H), jnp.float32),
        ],
    )(zp, y32, dis, b1.reshape(1, H))

    y2 = pl.pallas_call(
        _c2b_body,
        grid=(N // BM,),
        in_specs=[
            pl.BlockSpec((BM, H), lambda i: (i, 0)),
            pl.BlockSpec((2, H), lambda i: (0, 0)),
            pl.BlockSpec((1, H), lambda i: (0, 0)),
            pl.BlockSpec((1, H), lambda i: (0, 0)),
            pl.BlockSpec((BM, 1), lambda i: (i, 0)),
            pl.BlockSpec((H, CLS), lambda i: (0, 0)),
        ],
        out_specs=pl.BlockSpec((BM, CLS), lambda i: (i, 0)),
        out_shape=jax.ShapeDtypeStruct((N, CLS), jnp.float32),
    )(t, st, gamma.reshape(1, H), beta.reshape(1, H), dis, W2)

    z2p = _seg2_kernel(y2, src3, dst3, zeros_np2)

    out = pl.pallas_call(
        _c3_body,
        grid=(N // BM,),
        in_specs=[
            pl.BlockSpec((NC, BM, CLS), lambda i: (0, i, 0)),
            pl.BlockSpec((BM, CLS), lambda i: (i, 0)),
            pl.BlockSpec((BM, 1), lambda i: (i, 0)),
            pl.BlockSpec((1, CLS), lambda i: (0, 0)),
        ],
        out_specs=pl.BlockSpec((BM, CLS), lambda i: (i, 0)),
        out_shape=jax.ShapeDtypeStruct((N, CLS), jnp.float32),
    )(z2p, y2, dis, b2.reshape(1, CLS))

    return out


# speedup vs baseline: 30.1143x; 1.5535x over previous
import functools

import jax
import jax.numpy as jnp
from jax import lax
from jax.experimental import pallas as pl
from jax.experimental.pallas import tpu as pltpu
from jax.experimental.pallas import tpu_sc as plsc

N = 100000
E = 1600000
F = 165
H = 32
CLS = 2
EPS = 1e-5

NC = 2
NS = 16
NP = 100096
RP = NP // NS

CH = 625
NCHUNK = E // CH
NT_ALL = E // CH // NS // 2
NT_HALF = E // 2 // CH // NS // 2

BM = 2000

_mesh = plsc.VectorSubcoreMesh(core_axis_name="c", subcore_axis_name="s")
_sc_params = pltpu.CompilerParams(use_tc_tiling_on_sc=False)


def _wait_pair(src_hbm, dst_hbm, si, di, sem):
    pltpu.make_async_copy(src_hbm.at[0], si, sem).wait()
    pltpu.make_async_copy(dst_hbm.at[0], di, sem).wait()


def _fetch_pair(src_hbm, dst_hbm, si, di, chunk, sem):
    pltpu.async_copy(src_hbm.at[chunk], si, sem)
    pltpu.async_copy(dst_hbm.at[chunk], di, sem)


def _make_seg_kernel(split_features):
    nt = NT_ALL if split_features else NT_HALF

    @functools.partial(
        pl.kernel,
        out_type=jax.ShapeDtypeStruct((NC, NP, 16), jnp.float32),
        mesh=_mesh,
        compiler_params=_sc_params,
        scratch_types=[
            pltpu.VMEM((1, CH), jnp.int32),
            pltpu.VMEM((1, CH), jnp.int32),
            pltpu.VMEM((1, CH), jnp.int32),
            pltpu.VMEM((1, CH), jnp.int32),
            pltpu.VMEM((CH, 16), jnp.float32),
            pltpu.VMEM((CH, 16), jnp.float32),
            pltpu.VMEM_SHARED((NP, 16), jnp.float32),
            pltpu.SemaphoreType.DMA,
            pltpu.SemaphoreType.DMA,
            pltpu.SemaphoreType.DMA,
        ],
    )
    def _seg_kernel(y_hbm, src_hbm, dst_hbm, zeros_hbm, out_hbm,
                    si0, si1, di0, di1, rows0, rows1, acc_sh,
                    isem_a, isem_b, gsem):
        cid = lax.axis_index("c")
        sid = lax.axis_index("s")
        if split_features:
            ytab = y_hbm.at[cid]
            base = sid * (2 * nt)
        else:
            ytab = y_hbm
            base = cid * (NCHUNK // 2) + sid * (2 * nt)

        _fetch_pair(src_hbm, dst_hbm, si0, di0, base, isem_a)
        _fetch_pair(src_hbm, dst_hbm, si1, di1, base + 1, isem_b)
        pltpu.sync_copy(zeros_hbm, acc_sh.at[pl.ds(sid * RP, RP)])
        plsc.subcore_barrier()

        _wait_pair(src_hbm, dst_hbm, si0, di0, isem_a)
        pltpu.async_copy(ytab.at[si0.at[0]], rows0, gsem)

        @pl.loop(0, nt)
        def _(t):
            pltpu.make_async_copy(ytab.at[si0.at[0]], rows0, gsem).wait()
            _wait_pair(src_hbm, dst_hbm, si1, di1, isem_b)
            pltpu.async_copy(ytab.at[si1.at[0]], rows1, gsem)
            pltpu.sync_copy(rows0, acc_sh.at[di0.at[0]], add=True)

            @pl.when(t + 1 < nt)
            def _():
                _fetch_pair(src_hbm, dst_hbm, si0, di0, base + 2 * (t + 1),
                            isem_a)

            pltpu.make_async_copy(ytab.at[si1.at[0]], rows1, gsem).wait()
            pltpu.sync_copy(rows1, acc_sh.at[di1.at[0]], add=True)

            @pl.when(t + 1 < nt)
            def _():
                _fetch_pair(src_hbm, dst_hbm, si1, di1,
                            base + 2 * (t + 1) + 1, isem_b)
                _wait_pair(src_hbm, dst_hbm, si0, di0, isem_a)
                pltpu.async_copy(ytab.at[si0.at[0]], rows0, gsem)

        plsc.subcore_barrier()
        pltpu.sync_copy(acc_sh.at[pl.ds(sid * RP, RP)],
                        out_hbm.at[cid, pl.ds(sid * RP, RP)])

    return _seg_kernel


_seg32_kernel = _make_seg_kernel(split_features=True)
_seg16_kernel = _make_seg_kernel(split_features=False)


@functools.partial(
    pl.kernel,
    out_type=jax.ShapeDtypeStruct((NC, NP, 16), jnp.float32),
    mesh=_mesh,
    compiler_params=_sc_params,
    scratch_types=[
        pltpu.VMEM((1, CH), jnp.int32),
        pltpu.VMEM((1, CH), jnp.int32),
        pltpu.VMEM((CH, 16), jnp.float32),
        pltpu.VMEM_SHARED((NP, 16), jnp.float32),
        pltpu.SemaphoreType.DMA,
        pltpu.SemaphoreType.DMA,
    ],
)
def _deg_kernel(dst_hbm, ones_hbm, zeros_hbm, out_hbm,
                di0, di1, ones_v, acc_sh, isem_a, isem_b):
    cid = lax.axis_index("c")
    sid = lax.axis_index("s")
    base = cid * (NCHUNK // 2) + sid * (2 * NT_HALF)

    pltpu.async_copy(dst_hbm.at[base], di0, isem_a)
    pltpu.async_copy(dst_hbm.at[base + 1], di1, isem_b)
    pltpu.sync_copy(ones_hbm, ones_v)
    pltpu.sync_copy(zeros_hbm, acc_sh.at[pl.ds(sid * RP, RP)])
    plsc.subcore_barrier()

    @pl.loop(0, NT_HALF)
    def _(t):
        pltpu.make_async_copy(dst_hbm.at[0], di0, isem_a).wait()
        pltpu.sync_copy(ones_v, acc_sh.at[di0.at[0]], add=True)

        @pl.when(t + 1 < NT_HALF)
        def _():
            pltpu.async_copy(dst_hbm.at[base + 2 * (t + 1)], di0, isem_a)

        pltpu.make_async_copy(dst_hbm.at[0], di1, isem_b).wait()
        pltpu.sync_copy(ones_v, acc_sh.at[di1.at[0]], add=True)

        @pl.when(t + 1 < NT_HALF)
        def _():
            pltpu.async_copy(dst_hbm.at[base + 2 * (t + 1) + 1], di1, isem_b)

    plsc.subcore_barrier()
    pltpu.sync_copy(acc_sh.at[pl.ds(sid * RP, RP)],
                    out_hbm.at[cid, pl.ds(sid * RP, RP)])


def _m1_body(x_ref, w_ref, o_ref):
    o_ref[...] = jnp.dot(x_ref[...], w_ref[...],
                         preferred_element_type=jnp.float32)


def _s2_body(dp_ref, xw_ref, y3_ref, y32_ref, dis_ref):
    deg = dp_ref[0] + dp_ref[1] + 1.0
    dis16 = lax.rsqrt(deg)
    y = xw_ref[...] * jnp.concatenate([dis16, dis16], axis=1)
    y32_ref[...] = y
    dis_ref[...] = dis16[:, 0:1]
    y3_ref[0] = y[:, :16]
    y3_ref[1] = y[:, 16:]


def _c2a_body(zp_ref, y_ref, dis_ref, b1_ref, t_ref, st_ref):
    i = pl.program_id(0)
    z = jnp.concatenate([zp_ref[0], zp_ref[1]], axis=1)
    t = dis_ref[...] * (z + y_ref[...]) + b1_ref[...]
    t_ref[...] = t

    @pl.when(i == 0)
    def _():
        st_ref[...] = jnp.zeros_like(st_ref)

    st_ref[0:1, :] += jnp.sum(t, axis=0, keepdims=True)
    st_ref[1:2, :] += jnp.sum(t * t, axis=0, keepdims=True)


def _c2b_body(t_ref, st_ref, g_ref, be_ref, dis_ref, w2_ref, y2_ref):
    mean = st_ref[0:1, :] * (1.0 / N)
    var = st_ref[1:2, :] * (1.0 / N) - mean * mean
    inv = lax.rsqrt(var + EPS)
    h = (t_ref[...] - mean) * inv * g_ref[...] + be_ref[...]
    h = jnp.maximum(h, 0.0)
    y2 = jnp.dot(h, w2_ref[...], preferred_element_type=jnp.float32)
    y2 = y2 * dis_ref[...]
    y2_ref[...] = jnp.concatenate(
        [y2, jnp.zeros((y2.shape[0], 16 - CLS), jnp.float32)], axis=1)


def _c3_body(z2p_ref, y2_ref, dis_ref, b2_ref, o_ref):
    z2 = z2p_ref[0][:, :CLS] + z2p_ref[1][:, :CLS]
    o = dis_ref[...] * (z2 + y2_ref[:, :CLS]) + b2_ref[...]
    m = jnp.max(o, axis=1, keepdims=True)
    lse = m + jnp.log(jnp.sum(jnp.exp(o - m), axis=1, keepdims=True))
    o_ref[...] = o - lse


def kernel(x, edge_index, W1, b1, gamma, beta, W2, b2):
    ei = edge_index.astype(jnp.int32)
    src3 = ei[0].reshape(NCHUNK, 1, CH)
    dst3 = ei[1].reshape(NCHUNK, 1, CH)

    ones_ch = jnp.ones((CH, 16), jnp.float32)
    zeros_rp16 = jnp.zeros((RP, 16), jnp.float32)

    degp = _deg_kernel(dst3, ones_ch, zeros_rp16)

    xw = pl.pallas_call(
        _m1_body,
        grid=(N // BM,),
        in_specs=[
            pl.BlockSpec((BM, F), lambda i: (i, 0)),
            pl.BlockSpec((F, H), lambda i: (0, 0)),
        ],
        out_specs=pl.BlockSpec((BM, H), lambda i: (i, 0)),
        out_shape=jax.ShapeDtypeStruct((N, H), jnp.float32),
    )(x, W1)

    y3, y32, dis = pl.pallas_call(
        _s2_body,
        grid=(N // BM,),
        in_specs=[
            pl.BlockSpec((NC, BM, 16), lambda i: (0, i, 0)),
            pl.BlockSpec((BM, H), lambda i: (i, 0)),
        ],
        out_specs=[
            pl.BlockSpec((NC, BM, 16), lambda i: (0, i, 0)),
            pl.BlockSpec((BM, H), lambda i: (i, 0)),
            pl.BlockSpec((BM, 1), lambda i: (i, 0)),
        ],
        out_shape=[
            jax.ShapeDtypeStruct((NC, N, 16), jnp.float32),
            jax.ShapeDtypeStruct((N, H), jnp.float32),
            jax.ShapeDtypeStruct((N, 1), jnp.float32),
        ],
    )(degp, xw)

    zp = _seg32_kernel(y3, src3, dst3, zeros_rp16)

    t, st = pl.pallas_call(
        _c2a_body,
        grid=(N // BM,),
        in_specs=[
            pl.BlockSpec((NC, BM, 16), lambda i: (0, i, 0)),
            pl.BlockSpec((BM, H), lambda i: (i, 0)),
            pl.BlockSpec((BM, 1), lambda i: (i, 0)),
            pl.BlockSpec((1, H), lambda i: (0, 0)),
        ],
        out_specs=[
            pl.BlockSpec((BM, H), lambda i: (i, 0)),
            pl.BlockSpec((2, H), lambda i: (0, 0)),
        ],
        out_shape=[
            jax.ShapeDtypeStruct((N, H), jnp.float32),
            jax.ShapeDtypeStruct((2, H), jnp.float32),
        ],
    )(zp, y32, dis, b1.reshape(1, H))

    y2 = pl.pallas_call(
        _c2b_body,
        grid=(N // BM,),
        in_specs=[
            pl.BlockSpec((BM, H), lambda i: (i, 0)),
            pl.BlockSpec((2, H), lambda i: (0, 0)),
            pl.BlockSpec((1, H), lambda i: (0, 0)),
            pl.BlockSpec((1, H), lambda i: (0, 0)),
            pl.BlockSpec((BM, 1), lambda i: (i, 0)),
            pl.BlockSpec((H, CLS), lambda i: (0, 0)),
        ],
        out_specs=pl.BlockSpec((BM, 16), lambda i: (i, 0)),
        out_shape=jax.ShapeDtypeStruct((N, 16), jnp.float32),
    )(t, st, gamma.reshape(1, H), beta.reshape(1, H), dis, W2)

    z2p = _seg16_kernel(y2, src3, dst3, zeros_rp16)

    out = pl.pallas_call(
        _c3_body,
        grid=(N // BM,),
        in_specs=[
            pl.BlockSpec((NC, BM, 16), lambda i: (0, i, 0)),
            pl.BlockSpec((BM, 16), lambda i: (i, 0)),
            pl.BlockSpec((BM, 1), lambda i: (i, 0)),
            pl.BlockSpec((1, CLS), lambda i: (0, 0)),
        ],
        out_specs=pl.BlockSpec((BM, CLS), lambda i: (i, 0)),
        out_shape=jax.ShapeDtypeStruct((N, CLS), jnp.float32),
    )(z2p, y2, dis, b2.reshape(1, CLS))

    return out
